# baseline (speedup 1.0000x reference)
import functools

import jax
import jax.numpy as jnp
from jax import lax
from jax.experimental import pallas as pl
from jax.experimental.pallas import tpu as pltpu
from jax.experimental.pallas import tpu_sc as plsc

B = 4096
F = 13
L = 20
V = 100000
D = 32
NF = 512

_NC = 2
_NS = 16
_C = 256
_NCH = B // _C


def _prep_body(wover_ref, wdense_ref, bdense_ref, bover_ref,
               u_ref, g_ref, c0_ref):
    u = jnp.mean(wover_ref[...], axis=1, keepdims=True)
    u_ref[...] = u
    ud = u[:NF]
    g_ref[...] = jnp.dot(wdense_ref[...], ud,
                         preferred_element_type=jnp.float32)
    bd = jnp.reshape(bdense_ref[...], (NF, 1))
    c0 = jnp.sum(bd * ud) + jnp.mean(bover_ref[...])
    c0_ref[0] = c0


def _proj_body(t_ref, u_ref, p_ref):
    t = t_ref[0]
    uu = u_ref[0]
    p_ref[0, 0, :] = jnp.sum(t * uu, axis=0)


def _final_body(ff_ref, g_ref, part_ref, c0_ref, out_ref):
    z = jnp.dot(ff_ref[...], g_ref[...],
                preferred_element_type=jnp.float32)
    s = z[:, 0] + jnp.sum(part_ref[...], axis=0) + c0_ref[0]
    out_ref[...] = jax.nn.sigmoid(s)


def _sc_gather_body(p_hbm, q_hbm, idl_hbm, ids_hbm, w_hbm, out_hbm,
                    ptab_v, idx_v, wgt_v, out_v, sem_i, sem_w):
    cid = lax.axis_index("c")
    sid = lax.axis_index("s")
    wid = sid * _NC + cid

    def run(tab_src, idx_src, w_src, out_row):
        pltpu.sync_copy(tab_src, ptab_v)
        pltpu.make_async_copy(
            idx_src.at[:, pl.ds(0, _C)], idx_v.at[0], sem_i).start()
        if w_src is not None:
            pltpu.make_async_copy(
                w_src.at[:, pl.ds(0, _C)], wgt_v.at[0], sem_w).start()
        for cb in range(_NCH):
            buf = cb % 2
            pltpu.make_async_copy(
                idx_src.at[:, pl.ds(cb * _C, _C)], idx_v.at[buf],
                sem_i).wait()
            if w_src is not None:
                pltpu.make_async_copy(
                    w_src.at[:, pl.ds(cb * _C, _C)], wgt_v.at[buf],
                    sem_w).wait()
            if cb + 1 < _NCH:
                nxt = (cb + 1) * _C
                pltpu.make_async_copy(
                    idx_src.at[:, pl.ds(nxt, _C)], idx_v.at[1 - buf],
                    sem_i).start()
                if w_src is not None:
                    pltpu.make_async_copy(
                        w_src.at[:, pl.ds(nxt, _C)], wgt_v.at[1 - buf],
                        sem_w).start()

            def grp_body(i, carry, cb=cb, buf=buf):
                acc = jnp.zeros((16,), jnp.float32)
                if w_src is None:
                    for l in range(L):
                        idx16 = idx_v[buf, l, pl.ds(i * 16, 16)]
                        acc = acc + plsc.load_gather(ptab_v, [idx16])
                else:
                    for l in range(L):
                        idx16 = idx_v[buf, l, pl.ds(i * 16, 16)]
                        vals = plsc.load_gather(ptab_v, [idx16])
                        acc = acc + vals * wgt_v[buf, l, pl.ds(i * 16, 16)]
                out_v[pl.ds(cb * _C + i * 16, 16)] = acc
                return carry

            lax.fori_loop(0, _C // 16, grp_body, 0)
        pltpu.sync_copy(out_v, out_hbm.at[out_row])

    @pl.when(wid < F)
    def _unweighted():
        f = wid
        run(p_hbm.at[f, 0], idl_hbm.at[f], None, f)

    @pl.when(jnp.logical_and(wid >= F, wid < 2 * F))
    def _weighted():
        f = wid - F
        run(q_hbm.at[f, 0], ids_hbm.at[f], w_hbm.at[f], wid)


def kernel(float_features, idlist_indices, idscore_indices, idscore_weights,
           tables, weighted_tables, W_dense, b_dense, W_over, b_over):
    in_concat = W_over.shape[0]

    u, g, c0 = pl.pallas_call(
        _prep_body,
        out_shape=[
            jax.ShapeDtypeStruct((in_concat, 1), jnp.float32),
            jax.ShapeDtypeStruct((NF, 1), jnp.float32),
            jax.ShapeDtypeStruct((1,), jnp.float32),
        ],
        out_specs=[
            pl.BlockSpec(memory_space=pltpu.VMEM),
            pl.BlockSpec(memory_space=pltpu.VMEM),
            pl.BlockSpec(memory_space=pltpu.SMEM),
        ],
    )(W_over, W_dense, b_dense, b_over)

    u1 = u[:, 0]
    u2 = u1[NF:NF + F * D].reshape(F, D, 1)
    u3 = u1[NF + F * D:].reshape(F, D, 1)

    tables_t = jnp.swapaxes(tables, 1, 2)
    weighted_tables_t = jnp.swapaxes(weighted_tables, 1, 2)

    VB = 100352
    proj = pl.pallas_call(
        _proj_body,
        grid=(F, 1),
        in_specs=[
            pl.BlockSpec((1, D, VB), lambda f, v: (f, 0, v)),
            pl.BlockSpec((1, D, 1), lambda f, v: (f, 0, 0)),
        ],
        out_specs=pl.BlockSpec((1, 1, VB), lambda f, v: (f, 0, v)),
        out_shape=jax.ShapeDtypeStruct((F, 1, V), jnp.float32),
    )
    p3 = proj(tables_t, u2)
    q3 = proj(weighted_tables_t, u3)

    idl_t = jnp.swapaxes(idlist_indices, 1, 2)
    ids_t = jnp.swapaxes(idscore_indices, 1, 2)
    w_t = jnp.swapaxes(idscore_weights, 1, 2)

    mesh = plsc.VectorSubcoreMesh(core_axis_name="c", subcore_axis_name="s")
    sc_gather = functools.partial(
        pl.kernel,
        mesh=mesh,
        compiler_params=pltpu.CompilerParams(needs_layout_passes=False),
        out_type=jax.ShapeDtypeStruct((2 * F, B), jnp.float32),
        scratch_types=[
            pltpu.VMEM((V,), jnp.float32),
            pltpu.VMEM((2, L, _C), jnp.int32),
            pltpu.VMEM((2, L, _C), jnp.float32),
            pltpu.VMEM((B,), jnp.float32),
            pltpu.SemaphoreType.DMA,
            pltpu.SemaphoreType.DMA,
        ],
    )(_sc_gather_body)
    part = sc_gather(p3, q3, idl_t, ids_t, w_t)

    pred = pl.pallas_call(
        _final_body,
        in_specs=[
            pl.BlockSpec(memory_space=pltpu.VMEM),
            pl.BlockSpec(memory_space=pltpu.VMEM),
            pl.BlockSpec(memory_space=pltpu.VMEM),
            pl.BlockSpec(memory_space=pltpu.SMEM),
        ],
        out_shape=jax.ShapeDtypeStruct((B,), jnp.float32),
    )(float_features, g, part, c0)
    return pred


# speedup vs baseline: 1.0247x; 1.0247x over previous
import functools

import jax
import jax.numpy as jnp
from jax import lax
from jax.experimental import pallas as pl
from jax.experimental.pallas import tpu as pltpu
from jax.experimental.pallas import tpu_sc as plsc

B = 4096
F = 13
L = 20
V = 100000
D = 32
NF = 512

_NC = 2
_NS = 16
_C = 256
_NCH = B // _C


def _prep_body(wover_ref, wdense_ref, bdense_ref, bover_ref,
               u_ref, g_ref, c0_ref):
    u = jnp.mean(wover_ref[...], axis=1, keepdims=True)
    u_ref[...] = u
    ud = u[:NF]
    g_ref[...] = jnp.dot(wdense_ref[...], ud,
                         preferred_element_type=jnp.float32)
    bd = jnp.reshape(bdense_ref[...], (NF, 1))
    c0 = jnp.sum(bd * ud) + jnp.mean(bover_ref[...])
    c0_ref[0] = c0


def _proj_body(t_ref, wt_ref, u2_ref, u3_ref, p_ref, q_ref):
    t = t_ref[0]
    wt = wt_ref[0]
    u2 = u2_ref[0]
    u3 = u3_ref[0]
    p_ref[0, 0, :] = jnp.sum(t * u2, axis=0)
    q_ref[0, 0, :] = jnp.sum(wt * u3, axis=0)


def _final_body(ff_ref, g_ref, part_ref, c0_ref, out_ref):
    z = jnp.dot(ff_ref[...], g_ref[...],
                preferred_element_type=jnp.float32)
    s = z[:, 0] + jnp.sum(part_ref[...], axis=0) + c0_ref[0]
    out_ref[...] = jax.nn.sigmoid(s)


def _sc_gather_body(p_hbm, q_hbm, idl_hbm, ids_hbm, w_hbm, out_hbm,
                    ptab_v, idx_v, wgt_v, out_v, sem_i, sem_w):
    cid = lax.axis_index("c")
    sid = lax.axis_index("s")
    wid = sid * _NC + cid

    def run(tab_src, idx_src, w_src, out_row):
        pltpu.sync_copy(tab_src, ptab_v)
        pltpu.make_async_copy(
            idx_src.at[:, pl.ds(0, _C)], idx_v.at[0], sem_i).start()
        if w_src is not None:
            pltpu.make_async_copy(
                w_src.at[:, pl.ds(0, _C)], wgt_v.at[0], sem_w).start()
        for cb in range(_NCH):
            buf = cb % 2
            pltpu.make_async_copy(
                idx_src.at[:, pl.ds(cb * _C, _C)], idx_v.at[buf],
                sem_i).wait()
            if w_src is not None:
                pltpu.make_async_copy(
                    w_src.at[:, pl.ds(cb * _C, _C)], wgt_v.at[buf],
                    sem_w).wait()
            if cb + 1 < _NCH:
                nxt = (cb + 1) * _C
                pltpu.make_async_copy(
                    idx_src.at[:, pl.ds(nxt, _C)], idx_v.at[1 - buf],
                    sem_i).start()
                if w_src is not None:
                    pltpu.make_async_copy(
                        w_src.at[:, pl.ds(nxt, _C)], wgt_v.at[1 - buf],
                        sem_w).start()

            def grp_body(i, carry, cb=cb, buf=buf):
                acc = jnp.zeros((16,), jnp.float32)
                if w_src is None:
                    for l in range(L):
                        idx16 = idx_v[buf, l, pl.ds(i * 16, 16)]
                        acc = acc + plsc.load_gather(ptab_v, [idx16])
                else:
                    for l in range(L):
                        idx16 = idx_v[buf, l, pl.ds(i * 16, 16)]
                        vals = plsc.load_gather(ptab_v, [idx16])
                        acc = acc + vals * wgt_v[buf, l, pl.ds(i * 16, 16)]
                out_v[pl.ds(cb * _C + i * 16, 16)] = acc
                return carry

            lax.fori_loop(0, _C // 16, grp_body, 0)
        pltpu.sync_copy(out_v, out_hbm.at[out_row])

    @pl.when(wid < F)
    def _unweighted():
        f = wid
        run(p_hbm.at[f, 0], idl_hbm.at[f], None, f)

    @pl.when(jnp.logical_and(wid >= F, wid < 2 * F))
    def _weighted():
        f = wid - F
        run(q_hbm.at[f, 0], ids_hbm.at[f], w_hbm.at[f], wid)


def kernel(float_features, idlist_indices, idscore_indices, idscore_weights,
           tables, weighted_tables, W_dense, b_dense, W_over, b_over):
    in_concat = W_over.shape[0]

    u, g, c0 = pl.pallas_call(
        _prep_body,
        out_shape=[
            jax.ShapeDtypeStruct((in_concat, 1), jnp.float32),
            jax.ShapeDtypeStruct((NF, 1), jnp.float32),
            jax.ShapeDtypeStruct((1,), jnp.float32),
        ],
        out_specs=[
            pl.BlockSpec(memory_space=pltpu.VMEM),
            pl.BlockSpec(memory_space=pltpu.VMEM),
            pl.BlockSpec(memory_space=pltpu.SMEM),
        ],
    )(W_over, W_dense, b_dense, b_over)

    u1 = u[:, 0]
    u2 = u1[NF:NF + F * D].reshape(F, D, 1)
    u3 = u1[NF + F * D:].reshape(F, D, 1)

    tables_t = jnp.swapaxes(tables, 1, 2)
    weighted_tables_t = jnp.swapaxes(weighted_tables, 1, 2)

    VB = 50176
    nvb = pl.cdiv(V, VB)
    p3, q3 = pl.pallas_call(
        _proj_body,
        grid=(F, nvb),
        in_specs=[
            pl.BlockSpec((1, D, VB), lambda f, v: (f, 0, v)),
            pl.BlockSpec((1, D, VB), lambda f, v: (f, 0, v)),
            pl.BlockSpec((1, D, 1), lambda f, v: (f, 0, 0)),
            pl.BlockSpec((1, D, 1), lambda f, v: (f, 0, 0)),
        ],
        out_specs=[
            pl.BlockSpec((1, 1, VB), lambda f, v: (f, 0, v)),
            pl.BlockSpec((1, 1, VB), lambda f, v: (f, 0, v)),
        ],
        out_shape=[
            jax.ShapeDtypeStruct((F, 1, V), jnp.float32),
            jax.ShapeDtypeStruct((F, 1, V), jnp.float32),
        ],
    )(tables_t, weighted_tables_t, u2, u3)

    idl_t = jnp.swapaxes(idlist_indices, 1, 2)
    ids_t = jnp.swapaxes(idscore_indices, 1, 2)
    w_t = jnp.swapaxes(idscore_weights, 1, 2)

    mesh = plsc.VectorSubcoreMesh(core_axis_name="c", subcore_axis_name="s")
    sc_gather = functools.partial(
        pl.kernel,
        mesh=mesh,
        compiler_params=pltpu.CompilerParams(needs_layout_passes=False),
        out_type=jax.ShapeDtypeStruct((2 * F, B), jnp.float32),
        scratch_types=[
            pltpu.VMEM((V,), jnp.float32),
            pltpu.VMEM((2, L, _C), jnp.int32),
            pltpu.VMEM((2, L, _C), jnp.float32),
            pltpu.VMEM((B,), jnp.float32),
            pltpu.SemaphoreType.DMA,
            pltpu.SemaphoreType.DMA,
        ],
    )(_sc_gather_body)
    part = sc_gather(p3, q3, idl_t, ids_t, w_t)

    pred = pl.pallas_call(
        _final_body,
        in_specs=[
            pl.BlockSpec(memory_space=pltpu.VMEM),
            pl.BlockSpec(memory_space=pltpu.VMEM),
            pl.BlockSpec(memory_space=pltpu.VMEM),
            pl.BlockSpec(memory_space=pltpu.SMEM),
        ],
        out_shape=jax.ShapeDtypeStruct((B,), jnp.float32),
    )(float_features, g, part, c0)
    return pred
